# speedup vs baseline: 5.0434x; 5.0434x over previous
import jax
import jax.numpy as jnp
from jax.experimental import pallas as pl


_BLOCK_S = 512


def _copy_body(emb_ref, out_ref):
    blk = emb_ref[...]
    out_ref[...] = jnp.broadcast_to(blk[None, :, :], out_ref.shape)


def kernel(x, emb):
    batch, seq_len, d_model = x.shape
    grid = (seq_len // _BLOCK_S,)
    return pl.pallas_call(
        _copy_body,
        grid=grid,
        in_specs=[pl.BlockSpec((_BLOCK_S, d_model), lambda i: (i, 0))],
        out_specs=pl.BlockSpec((batch, _BLOCK_S, d_model), lambda i: (0, i, 0)),
        out_shape=jax.ShapeDtypeStruct((batch, seq_len, d_model), emb.dtype),
    )(emb[:seq_len])


# speedup vs baseline: 5.1829x; 1.0276x over previous
import jax
import jax.numpy as jnp
from jax.experimental import pallas as pl


_BLOCK_S = 1024


def _copy_body(emb_ref, out_ref):
    blk = emb_ref[...]
    out_ref[...] = jnp.broadcast_to(blk[None, :, :], out_ref.shape)


def kernel(x, emb):
    batch, seq_len, d_model = x.shape
    grid = (seq_len // _BLOCK_S,)
    return pl.pallas_call(
        _copy_body,
        grid=grid,
        in_specs=[pl.BlockSpec((_BLOCK_S, d_model), lambda i: (i, 0))],
        out_specs=pl.BlockSpec((batch, _BLOCK_S, d_model), lambda i: (0, i, 0)),
        out_shape=jax.ShapeDtypeStruct((batch, seq_len, d_model), emb.dtype),
    )(emb[:seq_len])
